# baseline (speedup 1.0000x reference)
import jax
import jax.numpy as jnp
from jax import lax
from jax.experimental import pallas as pl
from jax.experimental.pallas import tpu as pltpu
from jax.experimental.pallas import tpu_sc as plsc

VOCAB = 100000
SEQ = 2048
DIM = 128
BATCH = 16

NC = 2
NS = 16
NW = NC * NS
LANES = 16
VPR = DIM // LANES

PW = SEQ // NW
BPS = 2
STEPS = BATCH // BPS
RPS = BPS * PW
NBUF = 6
PRIME = 4
EARLY = PRIME * BPS


def _add_pos(rows_v, pos_v, k):
    for j in range(BPS):
        base = j * PW

        @plsc.parallel_loop(0, PW, step=1, unroll=8)
        def _(r):
            for u in range(VPR):
                off = u * LANES
                x = pos_v[r, pl.ds(off, LANES)]
                plsc.addupdate(rows_v.at[k, base + r, pl.ds(off, LANES)], x)


def _body(vals_hbm, tok_hbm, pos_hbm, out_hbm, idx_v, pos_v, rows_v,
          gsem, ssem, esem):
    cid = lax.axis_index("c")
    sid = lax.axis_index("s")
    wid = sid * NC + cid
    p0 = wid * PW

    def idx_cp(b, sem):
        return pltpu.async_copy(vals_hbm.at[pl.ds(b * SEQ + p0, PW)],
                                idx_v.at[b], sem)

    early_cps = [idx_cp(b, esem) for b in range(EARLY)]
    late_cps = [idx_cp(b, gsem) for b in range(EARLY, BATCH)]

    gathers = [[None] * BPS for _ in range(STEPS)]
    stores = [[None] * BPS for _ in range(STEPS)]

    def start_gather(s):
        k = s % NBUF
        for j in range(BPS):
            b = s * BPS + j
            gathers[s][j] = pltpu.async_copy(
                tok_hbm.at[idx_v.at[b]],
                rows_v.at[k, pl.ds(j * PW, PW)], gsem)

    def start_store(s):
        k = s % NBUF
        for j in range(BPS):
            b = s * BPS + j
            stores[s][j] = pltpu.async_copy(
                rows_v.at[k, pl.ds(j * PW, PW)],
                out_hbm.at[pl.ds(b * SEQ + p0, PW)], ssem)

    for cp in early_cps:
        cp.wait()
    for s in range(PRIME):
        start_gather(s)
    pltpu.sync_copy(pos_hbm.at[pl.ds(p0, PW)], pos_v)
    for cp in late_cps:
        cp.wait()

    for s in range(STEPS):
        k = s % NBUF
        for cp in gathers[s]:
            cp.wait()
        _add_pos(rows_v, pos_v, k)
        start_store(s)
        n = s + PRIME
        if n < STEPS:
            if n >= NBUF:
                for cp in stores[n - NBUF]:
                    cp.wait()
            start_gather(n)

    for s in range(STEPS - NBUF, STEPS):
        for cp in stores[s]:
            cp.wait()


@jax.jit
def kernel(values, tok_table, pos_table):
    vals = values.reshape(BATCH * SEQ).astype(jnp.int32)
    mesh = plsc.VectorSubcoreMesh(core_axis_name="c", subcore_axis_name="s")
    out = pl.kernel(
        _body,
        out_type=jax.ShapeDtypeStruct((BATCH * SEQ, DIM), jnp.float32),
        mesh=mesh,
        scratch_types=[
            pltpu.VMEM((BATCH, PW), jnp.int32),
            pltpu.VMEM((PW, DIM), jnp.float32),
            pltpu.VMEM((NBUF, RPS, DIM), jnp.float32),
            pltpu.SemaphoreType.DMA,
            pltpu.SemaphoreType.DMA,
            pltpu.SemaphoreType.DMA,
        ],
    )(vals, tok_table, pos_table)
    return out.reshape(BATCH, SEQ, DIM)


# speedup vs baseline: 1.1665x; 1.1665x over previous
import jax
import jax.numpy as jnp
from jax import lax
from jax.experimental import pallas as pl
from jax.experimental.pallas import tpu as pltpu
from jax.experimental.pallas import tpu_sc as plsc

VOCAB = 100000
SEQ = 2048
DIM = 128
BATCH = 16

NC = 2
NS = 16
NW = NC * NS
LANES = 16
VPR = DIM // LANES

PW = SEQ // NW
BPS = 2
STEPS = BATCH // BPS
RPS = BPS * PW
NBUF = 4


def _add_pos(rows_v, pos_v, k):
    @plsc.parallel_loop(0, RPS, step=1, unroll=4)
    def _(r):
        prow = lax.rem(r, PW)
        for u in range(VPR):
            off = u * LANES
            x = pos_v[prow, pl.ds(off, LANES)]
            plsc.addupdate(rows_v.at[k, r, pl.ds(off, LANES)], x)


def _body(vals_hbm, tok_hbm, pos_hbm, out_hbm, idx_v, pos_v, rows_v,
          gsem, ssem):
    cid = lax.axis_index("c")
    sid = lax.axis_index("s")
    wid = sid * NC + cid
    p0 = wid * PW

    idx_cps = [
        pltpu.async_copy(vals_hbm.at[pl.ds(b * SEQ + p0, PW)],
                         idx_v.at[b], gsem)
        for b in range(BATCH)
    ]
    pltpu.sync_copy(pos_hbm.at[pl.ds(p0, PW)], pos_v)
    for cp in idx_cps:
        cp.wait()

    gathers = [None] * STEPS
    stores = [None] * STEPS

    def start_gather(s):
        k = s % NBUF
        cps = []
        for j in range(BPS):
            b = s * BPS + j
            cps.append(pltpu.async_copy(
                tok_hbm.at[idx_v.at[b]],
                rows_v.at[k, pl.ds(j * PW, PW)], gsem))
        gathers[s] = cps

    def start_store(s):
        k = s % NBUF
        cps = []
        for j in range(BPS):
            b = s * BPS + j
            cps.append(pltpu.async_copy(
                rows_v.at[k, pl.ds(j * PW, PW)],
                out_hbm.at[pl.ds(b * SEQ + p0, PW)], ssem))
        stores[s] = cps

    for s in range(NBUF - 1):
        start_gather(s)

    for s in range(STEPS):
        k = s % NBUF
        for cp in gathers[s]:
            cp.wait()
        _add_pos(rows_v, pos_v, k)
        start_store(s)
        ns = s + NBUF - 1
        if ns < STEPS:
            if s >= 1:
                for cp in stores[s - 1]:
                    cp.wait()
            start_gather(ns)

    for s in range(max(0, STEPS - NBUF), STEPS):
        for cp in stores[s]:
            cp.wait()


@jax.jit
def kernel(values, tok_table, pos_table):
    vals = values.reshape(BATCH * SEQ).astype(jnp.int32)
    mesh = plsc.VectorSubcoreMesh(core_axis_name="c", subcore_axis_name="s")
    out = pl.kernel(
        _body,
        out_type=jax.ShapeDtypeStruct((BATCH * SEQ, DIM), jnp.float32),
        mesh=mesh,
        scratch_types=[
            pltpu.VMEM((BATCH, PW), jnp.int32),
            pltpu.VMEM((PW, DIM), jnp.float32),
            pltpu.VMEM((NBUF, RPS, DIM), jnp.float32),
            pltpu.SemaphoreType.DMA,
            pltpu.SemaphoreType.DMA,
        ],
    )(vals, tok_table, pos_table)
    return out.reshape(BATCH, SEQ, DIM)


# speedup vs baseline: 2.6860x; 2.3027x over previous
import jax
import jax.numpy as jnp
from jax import lax
from jax.experimental import pallas as pl
from jax.experimental.pallas import tpu as pltpu
from jax.experimental.pallas import tpu_sc as plsc

VOCAB = 100000
SEQ = 2048
DIM = 128
BATCH = 16

NC = 2
NS = 16
NW = NC * NS
LANES = 16
VPR = DIM // LANES

PW = SEQ // NW
BPS = 2
STEPS = BATCH // BPS
RPS = BPS * PW
NBUF = 4


def _add_pos(rows_v, pos_v, k):
    @plsc.parallel_loop(0, PW, step=1, unroll=4)
    def _(r):
        for j in range(BPS):
            for u in range(VPR):
                off = u * LANES
                x = pos_v[r, pl.ds(off, LANES)]
                plsc.addupdate(rows_v.at[k, j * PW + r, pl.ds(off, LANES)], x)


def _body(vals_hbm, tok_hbm, pos_hbm, out_hbm, idx_v, pos_v, rows_v,
          gsem, ssem):
    cid = lax.axis_index("c")
    sid = lax.axis_index("s")
    wid = sid * NC + cid
    p0 = wid * PW

    idx_cps = [
        pltpu.async_copy(vals_hbm.at[pl.ds(b * SEQ + p0, PW)],
                         idx_v.at[b], gsem)
        for b in range(BATCH)
    ]
    pltpu.sync_copy(pos_hbm.at[pl.ds(p0, PW)], pos_v)
    for cp in idx_cps:
        cp.wait()

    gathers = [None] * STEPS
    stores = [None] * STEPS

    def start_gather(s):
        k = s % NBUF
        cps = []
        for j in range(BPS):
            b = s * BPS + j
            cps.append(pltpu.async_copy(
                tok_hbm.at[idx_v.at[b]],
                rows_v.at[k, pl.ds(j * PW, PW)], gsem))
        gathers[s] = cps

    def start_store(s):
        k = s % NBUF
        cps = []
        for j in range(BPS):
            b = s * BPS + j
            cps.append(pltpu.async_copy(
                rows_v.at[k, pl.ds(j * PW, PW)],
                out_hbm.at[pl.ds(b * SEQ + p0, PW)], ssem))
        stores[s] = cps

    for s in range(NBUF - 1):
        start_gather(s)

    for s in range(STEPS):
        k = s % NBUF
        for cp in gathers[s]:
            cp.wait()
        _add_pos(rows_v, pos_v, k)
        start_store(s)
        ns = s + NBUF - 1
        if ns < STEPS:
            if s >= 1:
                for cp in stores[s - 1]:
                    cp.wait()
            start_gather(ns)

    for s in range(max(0, STEPS - NBUF), STEPS):
        for cp in stores[s]:
            cp.wait()


@jax.jit
def kernel(values, tok_table, pos_table):
    vals = values.reshape(BATCH * SEQ).astype(jnp.int32)
    mesh = plsc.VectorSubcoreMesh(core_axis_name="c", subcore_axis_name="s")
    out = pl.kernel(
        _body,
        out_type=jax.ShapeDtypeStruct((BATCH * SEQ, DIM), jnp.float32),
        mesh=mesh,
        scratch_types=[
            pltpu.VMEM((BATCH, PW), jnp.int32),
            pltpu.VMEM((PW, DIM), jnp.float32),
            pltpu.VMEM((NBUF, RPS, DIM), jnp.float32),
            pltpu.SemaphoreType.DMA,
            pltpu.SemaphoreType.DMA,
        ],
    )(vals, tok_table, pos_table)
    return out.reshape(BATCH, SEQ, DIM)
